# baseline (speedup 1.0000x reference)
import functools
import math

import jax
import jax.numpy as jnp
from jax import lax
from jax.experimental import pallas as pl
from jax.experimental.pallas import tpu as pltpu
from jax.experimental.pallas import tpu_sc as plsc

_D = 128
_H = 16
_SCALE = math.sqrt(float(_D))
_N = 1024 * 50
_NC = 2
_NS = 16
_NW = _NC * _NS
_PER_W = _N // _NW
_CHUNK = 128


def _chunk_list():
    out = []
    off = 0
    while off < _PER_W:
        sz = min(_CHUNK, _PER_W - off)
        out.append((off, sz))
        off += sz
    return out


def _scale_body(w_ref, o_ref):
    o_ref[:] = w_ref[:] * _SCALE


_mesh = plsc.VectorSubcoreMesh(core_axis_name="c", subcore_axis_name="s")


@functools.partial(
    pl.kernel,
    out_type=jax.ShapeDtypeStruct((_N, _H, _D), jnp.float32),
    mesh=_mesh,
    scratch_types=[
        pltpu.VMEM((_PER_W,), jnp.int32),
        pltpu.VMEM((_CHUNK, 1, _D), jnp.float32),
        pltpu.VMEM((_CHUNK, 1, _D), jnp.float32),
        pltpu.SemaphoreType.DMA,
        pltpu.SemaphoreType.DMA,
        pltpu.SemaphoreType.DMA,
    ],
)
def _sc_lookup(table_hbm, idx_hbm, out_hbm, idx_v, buf0, buf1, gsem, wsem0, wsem1):
    wid = lax.axis_index("s") * _NC + lax.axis_index("c")
    base = wid * _PER_W
    pltpu.sync_copy(idx_hbm.at[pl.ds(base, _PER_W)], idx_v)

    bufs = (buf0, buf1)
    wsems = (wsem0, wsem1)
    chunks = _chunk_list()
    pending = [None, None]

    g = pltpu.async_copy(
        table_hbm.at[idx_v.at[pl.ds(chunks[0][0], chunks[0][1])]],
        buf0.at[pl.ds(0, chunks[0][1])], gsem)
    for i, (off, sz) in enumerate(chunks):
        slot = i % 2
        g.wait()
        if i + 1 < len(chunks):
            noff, nsz = chunks[i + 1]
            nslot = (i + 1) % 2
            if pending[nslot] is not None:
                for w in pending[nslot]:
                    w.wait()
                pending[nslot] = None
            g = pltpu.async_copy(
                table_hbm.at[idx_v.at[pl.ds(noff, nsz)]],
                bufs[nslot].at[pl.ds(0, nsz)], gsem)
        pending[slot] = [
            pltpu.async_copy(
                bufs[slot].at[pl.ds(0, sz)],
                out_hbm.at[pl.ds(base + off, sz), pl.ds(h, 1)],
                wsems[slot])
            for h in range(_H)
        ]
    for ws in pending:
        if ws is not None:
            for w in ws:
                w.wait()


def kernel(inputs, W_v):
    W_s = pl.pallas_call(
        _scale_body,
        out_shape=jax.ShapeDtypeStruct(W_v.shape, W_v.dtype),
    )(W_v)
    table = W_s.reshape(W_v.shape[0], 1, _D)
    idx = inputs.reshape(-1)
    out = _sc_lookup(table, idx)
    return out.reshape(1, inputs.shape[0], inputs.shape[1], _H * _D)


# speedup vs baseline: 1.6378x; 1.6378x over previous
import functools
import math

import jax
import jax.numpy as jnp
from jax import lax
from jax.experimental import pallas as pl
from jax.experimental.pallas import tpu as pltpu
from jax.experimental.pallas import tpu_sc as plsc

_D = 128
_H = 16
_SCALE = math.sqrt(float(_D))
_B = 1024
_L = 50
_NC = 2
_NS = 16
_NW = _NC * _NS
_BPW = _B // _NW


def _scale_body(w_ref, o_ref):
    o_ref[:] = w_ref[:] * _SCALE


_mesh = plsc.VectorSubcoreMesh(core_axis_name="c", subcore_axis_name="s")


@functools.partial(
    pl.kernel,
    out_type=jax.ShapeDtypeStruct((1, _B, _L, _H * _D), jnp.float32),
    mesh=_mesh,
    scratch_types=[
        pltpu.VMEM((_BPW, _L), jnp.int32),
        pltpu.VMEM((_L, _D), jnp.float32),
        pltpu.VMEM((_L, _D), jnp.float32),
        pltpu.SemaphoreType.DMA,
        pltpu.SemaphoreType.DMA,
        pltpu.SemaphoreType.DMA,
    ],
    compiler_params=pltpu.CompilerParams(use_tc_tiling_on_sc=True),
)
def _sc_lookup(table_hbm, idx_hbm, out_hbm, idx_v, buf0, buf1, gsem, wsem0, wsem1):
    wid = lax.axis_index("s") * _NC + lax.axis_index("c")
    b0 = wid * _BPW
    pltpu.sync_copy(idx_hbm.at[pl.ds(b0, _BPW)], idx_v)

    bufs = (buf0, buf1)
    wsems = (wsem0, wsem1)
    pending = [None, None]

    g = pltpu.async_copy(table_hbm.at[idx_v.at[0]], buf0, gsem)
    for i in range(_BPW):
        slot = i % 2
        g.wait()
        if i + 1 < _BPW:
            nslot = (i + 1) % 2
            if pending[nslot] is not None:
                for w in pending[nslot]:
                    w.wait()
                pending[nslot] = None
            g = pltpu.async_copy(table_hbm.at[idx_v.at[i + 1]], bufs[nslot], gsem)
        pending[slot] = [
            pltpu.async_copy(
                bufs[slot],
                out_hbm.at[0, b0 + i, pl.ds(0, _L), pl.ds(h * _D, _D)],
                wsems[slot])
            for h in range(_H)
        ]
    for ws in pending:
        if ws is not None:
            for w in ws:
                w.wait()


def kernel(inputs, W_v):
    W_s = pl.pallas_call(
        _scale_body,
        out_shape=jax.ShapeDtypeStruct(W_v.shape, W_v.dtype),
    )(W_v)
    return _sc_lookup(W_s, inputs)


# speedup vs baseline: 4.0765x; 2.4891x over previous
import functools
import math

import jax
import jax.numpy as jnp
from jax import lax
from jax.experimental import pallas as pl
from jax.experimental.pallas import tpu as pltpu
from jax.experimental.pallas import tpu_sc as plsc

_D = 128
_H = 16
_SCALE = math.sqrt(float(_D))
_B = 1024
_L = 50
_NC = 2
_NS = 16
_NW = _NC * _NS
_BPW = _B // _NW


def _scale_body(w_ref, o_ref):
    o_ref[:] = w_ref[:] * _SCALE


_mesh = plsc.VectorSubcoreMesh(core_axis_name="c", subcore_axis_name="s")


@functools.partial(
    pl.kernel,
    out_type=jax.ShapeDtypeStruct((_L, _B, _H * _D), jnp.float32),
    mesh=_mesh,
    scratch_types=[
        pltpu.VMEM((_L, _BPW), jnp.int32),
        pltpu.VMEM((_BPW, _D), jnp.float32),
        pltpu.VMEM((_BPW, _D), jnp.float32),
        pltpu.SemaphoreType.DMA,
        pltpu.SemaphoreType.DMA,
        pltpu.SemaphoreType.DMA,
    ],
    compiler_params=pltpu.CompilerParams(use_tc_tiling_on_sc=True),
)
def _sc_lookup(table_hbm, idx_hbm, out_hbm, idx_v, buf0, buf1, gsem, wsem0, wsem1):
    wid = lax.axis_index("s") * _NC + lax.axis_index("c")
    b0 = wid * _BPW
    pltpu.sync_copy(idx_hbm.at[wid], idx_v)

    bufs = (buf0, buf1)
    wsems = (wsem0, wsem1)
    pending = [None, None]

    g = pltpu.async_copy(table_hbm.at[idx_v.at[0]], buf0, gsem)
    for l in range(_L):
        slot = l % 2
        g.wait()
        if l + 1 < _L:
            nslot = (l + 1) % 2
            if pending[nslot] is not None:
                for w in pending[nslot]:
                    w.wait()
                pending[nslot] = None
            g = pltpu.async_copy(table_hbm.at[idx_v.at[l + 1]], bufs[nslot], gsem)
        pending[slot] = [
            pltpu.async_copy(
                bufs[slot],
                out_hbm.at[l, pl.ds(b0, _BPW), pl.ds(h * _D, _D)],
                wsems[slot])
            for h in range(_H)
        ]
    for ws in pending:
        if ws is not None:
            for w in ws:
                w.wait()


def kernel(inputs, W_v):
    W_s = pl.pallas_call(
        _scale_body,
        out_shape=jax.ShapeDtypeStruct(W_v.shape, W_v.dtype),
    )(W_v)
    idx3 = inputs.T.reshape(_L, _NW, _BPW).transpose(1, 0, 2)
    out = _sc_lookup(W_s, idx3)
    return jnp.transpose(out, (1, 0, 2))[None]


# speedup vs baseline: 5.6380x; 1.3830x over previous
import functools
import math

import jax
import jax.numpy as jnp
from jax import lax
from jax.experimental import pallas as pl
from jax.experimental.pallas import tpu as pltpu
from jax.experimental.pallas import tpu_sc as plsc

_D = 128
_H = 16
_SCALE = math.sqrt(float(_D))
_B = 1024
_L = 50
_NC = 2
_NS = 16
_NW = _NC * _NS
_BPW = _B // _NW


def _scale_body(w_ref, o_ref):
    o_ref[:] = w_ref[:] * _SCALE


_mesh = plsc.VectorSubcoreMesh(core_axis_name="c", subcore_axis_name="s")


@functools.partial(
    pl.kernel,
    out_type=jax.ShapeDtypeStruct((_L, _B, _H * _D), jnp.float32),
    mesh=_mesh,
    scratch_types=[
        pltpu.VMEM((_L, _BPW), jnp.int32),
        pltpu.VMEM((_BPW, _D), jnp.float32),
        pltpu.VMEM((_BPW, _D), jnp.float32),
        pltpu.VMEM_SHARED((130, _D), jnp.float32),
        pltpu.SemaphoreType.DMA,
        pltpu.SemaphoreType.DMA,
        pltpu.SemaphoreType.DMA,
    ],
    compiler_params=pltpu.CompilerParams(use_tc_tiling_on_sc=True),
)
def _sc_lookup(table_hbm, idx_hbm, out_hbm, idx_v, buf0, buf1, tshared,
               gsem, wsem0, wsem1):
    wid = lax.axis_index("s") * _NC + lax.axis_index("c")
    b0 = wid * _BPW
    @pl.when(lax.axis_index("s") == 0)
    def _():
        pltpu.sync_copy(table_hbm, tshared)
    pltpu.sync_copy(idx_hbm.at[wid], idx_v)
    plsc.subcore_barrier()

    bufs = (buf0, buf1)
    wsems = (wsem0, wsem1)
    pending = [None, None]

    g = pltpu.async_copy(tshared.at[idx_v.at[0]], buf0, gsem)
    for l in range(_L):
        slot = l % 2
        g.wait()
        if l + 1 < _L:
            nslot = (l + 1) % 2
            if pending[nslot] is not None:
                for w in pending[nslot]:
                    w.wait()
                pending[nslot] = None
            g = pltpu.async_copy(tshared.at[idx_v.at[l + 1]], bufs[nslot], gsem)
        pending[slot] = [
            pltpu.async_copy(
                bufs[slot],
                out_hbm.at[l, pl.ds(b0, _BPW), pl.ds(h * _D, _D)],
                wsems[slot])
            for h in range(_H)
        ]
    for ws in pending:
        if ws is not None:
            for w in ws:
                w.wait()


def kernel(inputs, W_v):
    W_s = pl.pallas_call(
        _scale_body,
        out_shape=jax.ShapeDtypeStruct(W_v.shape, W_v.dtype),
    )(W_v)
    idx3 = inputs.T.reshape(_L, _NW, _BPW).transpose(1, 0, 2)
    out = _sc_lookup(W_s, idx3)
    return jnp.transpose(out, (1, 0, 2))[None]


# speedup vs baseline: 5.6997x; 1.0109x over previous
import functools
import math

import jax
import jax.numpy as jnp
from jax import lax
from jax.experimental import pallas as pl
from jax.experimental.pallas import tpu as pltpu
from jax.experimental.pallas import tpu_sc as plsc

_D = 128
_H = 16
_SCALE = math.sqrt(float(_D))
_B = 1024
_L = 50
_NC = 2
_NS = 16
_NW = _NC * _NS
_BPW = _B // _NW


def _scale_body(w_ref, o_ref):
    o_ref[:] = w_ref[:] * _SCALE


_LB = 2
_NI = _L // _LB

_mesh = plsc.VectorSubcoreMesh(core_axis_name="c", subcore_axis_name="s")


@functools.partial(
    pl.kernel,
    out_type=jax.ShapeDtypeStruct((_L, _B, _H * _D), jnp.float32),
    mesh=_mesh,
    scratch_types=[
        pltpu.VMEM((_L, _BPW), jnp.int32),
        pltpu.VMEM((_LB, _BPW, _D), jnp.float32),
        pltpu.VMEM((_LB, _BPW, _D), jnp.float32),
        pltpu.VMEM_SHARED((130, _D), jnp.float32),
        pltpu.SemaphoreType.DMA,
        pltpu.SemaphoreType.DMA,
        pltpu.SemaphoreType.DMA,
    ],
    compiler_params=pltpu.CompilerParams(use_tc_tiling_on_sc=True),
)
def _sc_lookup(table_hbm, idx_hbm, out_hbm, idx_v, buf0, buf1, tshared,
               gsem, wsem0, wsem1):
    wid = lax.axis_index("s") * _NC + lax.axis_index("c")
    b0 = wid * _BPW
    @pl.when(lax.axis_index("s") == 0)
    def _():
        pltpu.sync_copy(table_hbm, tshared)
    pltpu.sync_copy(idx_hbm.at[wid], idx_v)
    plsc.subcore_barrier()

    bufs = (buf0, buf1)
    wsems = (wsem0, wsem1)
    pending = [None, None]

    cur_g = [pltpu.async_copy(tshared.at[idx_v.at[j]], buf0.at[j], gsem)
             for j in range(_LB)]
    for i in range(_NI):
        slot = i % 2
        for g in cur_g:
            g.wait()
        if i + 1 < _NI:
            nslot = (i + 1) % 2
            if pending[nslot] is not None:
                for w in pending[nslot]:
                    w.wait()
                pending[nslot] = None
            cur_g = [
                pltpu.async_copy(tshared.at[idx_v.at[(i + 1) * _LB + j]],
                                 bufs[nslot].at[j], gsem)
                for j in range(_LB)
            ]
        pending[slot] = [
            pltpu.async_copy(
                bufs[slot],
                out_hbm.at[pl.ds(i * _LB, _LB), pl.ds(b0, _BPW),
                           pl.ds(h * _D, _D)],
                wsems[slot])
            for h in range(_H)
        ]
    for ws in pending:
        if ws is not None:
            for w in ws:
                w.wait()


def kernel(inputs, W_v):
    W_s = pl.pallas_call(
        _scale_body,
        out_shape=jax.ShapeDtypeStruct(W_v.shape, W_v.dtype),
    )(W_v)
    idx3 = inputs.T.reshape(_L, _NW, _BPW).transpose(1, 0, 2)
    out = _sc_lookup(W_s, idx3)
    return jnp.transpose(out, (1, 0, 2))[None]


# speedup vs baseline: 5.7681x; 1.0120x over previous
import functools
import math

import jax
import jax.numpy as jnp
from jax import lax
from jax.experimental import pallas as pl
from jax.experimental.pallas import tpu as pltpu
from jax.experimental.pallas import tpu_sc as plsc

_D = 128
_H = 16
_SCALE = math.sqrt(float(_D))
_B = 1024
_L = 50
_NC = 2
_NS = 16
_NW = _NC * _NS
_BPW = _B // _NW


def _scale_body(w_ref, o_ref):
    o_ref[:] = w_ref[:] * _SCALE


_LB = 10
_NI = _L // _LB

_mesh = plsc.VectorSubcoreMesh(core_axis_name="c", subcore_axis_name="s")


@functools.partial(
    pl.kernel,
    out_type=jax.ShapeDtypeStruct((_L, _B, _H * _D), jnp.float32),
    mesh=_mesh,
    scratch_types=[
        pltpu.VMEM((_L, _BPW), jnp.int32),
        pltpu.VMEM((_LB, _BPW, _D), jnp.float32),
        pltpu.VMEM((_LB, _BPW, _D), jnp.float32),
        pltpu.VMEM_SHARED((130, _D), jnp.float32),
        pltpu.SemaphoreType.DMA,
        pltpu.SemaphoreType.DMA,
        pltpu.SemaphoreType.DMA,
    ],
    compiler_params=pltpu.CompilerParams(use_tc_tiling_on_sc=True),
)
def _sc_lookup(table_hbm, idx_hbm, out_hbm, idx_v, buf0, buf1, tshared,
               gsem, wsem0, wsem1):
    wid = lax.axis_index("s") * _NC + lax.axis_index("c")
    b0 = wid * _BPW
    @pl.when(lax.axis_index("s") == 0)
    def _():
        pltpu.sync_copy(table_hbm, tshared)
    pltpu.sync_copy(idx_hbm.at[wid], idx_v)
    plsc.subcore_barrier()

    bufs = (buf0, buf1)
    wsems = (wsem0, wsem1)
    pending = [None, None]

    cur_g = [pltpu.async_copy(tshared.at[idx_v.at[j]], buf0.at[j], gsem)
             for j in range(_LB)]
    for i in range(_NI):
        slot = i % 2
        for g in cur_g:
            g.wait()
        if i + 1 < _NI:
            nslot = (i + 1) % 2
            if pending[nslot] is not None:
                for w in pending[nslot]:
                    w.wait()
                pending[nslot] = None
            cur_g = [
                pltpu.async_copy(tshared.at[idx_v.at[(i + 1) * _LB + j]],
                                 bufs[nslot].at[j], gsem)
                for j in range(_LB)
            ]
        pending[slot] = [
            pltpu.async_copy(
                bufs[slot],
                out_hbm.at[pl.ds(i * _LB, _LB), pl.ds(b0, _BPW),
                           pl.ds(h * _D, _D)],
                wsems[slot])
            for h in range(_H)
        ]
    for ws in pending:
        if ws is not None:
            for w in ws:
                w.wait()


def kernel(inputs, W_v):
    W_s = pl.pallas_call(
        _scale_body,
        out_shape=jax.ShapeDtypeStruct(W_v.shape, W_v.dtype),
    )(W_v)
    idx3 = inputs.T.reshape(_L, _NW, _BPW).transpose(1, 0, 2)
    out = _sc_lookup(W_s, idx3)
    return jnp.transpose(out, (1, 0, 2))[None]
